# Initial kernel scaffold; baseline (speedup 1.0000x reference)
import jax
import jax.numpy as jnp
from jax.experimental import pallas as pl


def kernel(word_encodings, src_floors, tgt_floors, emb_table, W, b):
    raise NotImplementedError("write your pallas kernel here")


# speedup vs baseline: 1.0826x; 1.0826x over previous
import functools

import jax
import jax.numpy as jnp
from jax import lax
from jax.experimental import pallas as pl
from jax.experimental.pallas import tpu as pltpu
from jax.experimental.pallas import tpu_sc as plsc

_INPUT_DIM = 1024
_EMBED_DIM = 32
_HIST_LEN = 50
_PAD_HIST = 64
_SEQ_TILE = 256


_ROW_PAD = 128


def _floor_gather_sc(src_pad, tgt_pad, table_flipped):
    n = src_pad.shape[0]
    n_chunks = n // 16
    mesh = plsc.VectorSubcoreMesh(core_axis_name="c", subcore_axis_name="s")

    @functools.partial(
        pl.kernel,
        mesh=mesh,
        out_type=jax.ShapeDtypeStruct((n, _ROW_PAD), jnp.float32),
        scratch_types=[
            pltpu.VMEM((16,), jnp.int32),
            pltpu.VMEM((16,), jnp.int32),
            pltpu.VMEM((16,), jnp.int32),
            pltpu.VMEM((16, _ROW_PAD), jnp.float32),
            pltpu.SemaphoreType.DMA,
        ],
    )
    def gather_kernel(src_hbm, tgt_hbm, table_hbm, out_hbm,
                      src_v, tgt_v, idx_v, rows_v, sem):
        wid = lax.axis_index("s") * 2 + lax.axis_index("c")

        @pl.when(wid < n_chunks)
        def _():
            base = wid * 16
            pltpu.sync_copy(src_hbm.at[pl.ds(base, 16)], src_v)
            pltpu.sync_copy(tgt_hbm.at[pl.ds(base, 16)], tgt_v)
            idx_v[...] = lax.bitwise_xor(src_v[...], tgt_v[...])
            pltpu.async_copy(table_hbm.at[idx_v], rows_v, sem).wait()
            pltpu.sync_copy(rows_v, out_hbm.at[pl.ds(base, 16)])

    return gather_kernel(src_pad, tgt_pad, table_flipped)


def _proj_body(x_ref, w1_ref, w2_ref, fv_ref, b_ref, o_ref):
    bias = jnp.dot(fv_ref[0], w2_ref[...],
                   preferred_element_type=jnp.float32) + b_ref[...]
    o_ref[...] = (jnp.dot(x_ref[0], w1_ref[...],
                          preferred_element_type=jnp.float32) + bias)[None]


def kernel(word_encodings, src_floors, tgt_floors, emb_table, W, b):
    B, S, D = word_encodings.shape
    hv = _HIST_LEN * _EMBED_DIM

    pad = _PAD_HIST - _HIST_LEN
    src_pad = jnp.pad(src_floors.astype(jnp.int32), ((0, 0), (0, pad))).reshape(-1)
    tgt_pad = jnp.pad(tgt_floors.astype(jnp.int32), ((0, 0), (0, pad))).reshape(-1)
    table_flipped = jnp.pad(emb_table[::-1], ((0, 0), (0, _ROW_PAD - _EMBED_DIM)))

    rows = _floor_gather_sc(src_pad, tgt_pad, table_flipped)
    fv = rows.reshape(B, _PAD_HIST, _ROW_PAD)[:, :_HIST_LEN, :_EMBED_DIM]
    fv = fv.reshape(B, 1, hv)

    grid = (B, S // _SEQ_TILE)
    out = pl.pallas_call(
        _proj_body,
        grid=grid,
        in_specs=[
            pl.BlockSpec((1, _SEQ_TILE, D), lambda i, j: (i, j, 0)),
            pl.BlockSpec((D, D), lambda i, j: (0, 0)),
            pl.BlockSpec((hv, D), lambda i, j: (0, 0)),
            pl.BlockSpec((1, 1, hv), lambda i, j: (i, 0, 0)),
            pl.BlockSpec((1, D), lambda i, j: (0, 0)),
        ],
        out_specs=pl.BlockSpec((1, _SEQ_TILE, D), lambda i, j: (i, j, 0)),
        out_shape=jax.ShapeDtypeStruct((B, S, D), jnp.float32),
        compiler_params=pltpu.CompilerParams(
            dimension_semantics=("parallel", "parallel"),
        ),
    )(word_encodings, W[:D], W[D:], fv, b.reshape(1, D))
    return out


# speedup vs baseline: 1.1742x; 1.0846x over previous
import functools

import jax
import jax.numpy as jnp
from jax import lax
from jax.experimental import pallas as pl
from jax.experimental.pallas import tpu as pltpu
from jax.experimental.pallas import tpu_sc as plsc

_EMBED_DIM = 32
_SEQ_TILE = 256


_ROW_PAD = 128


def _floor_gather_sc(src_flat, tgt_flat, table_padded):
    n = src_flat.shape[0]
    n_chunks = (n + 15) // 16
    last_base = n - 16
    mesh = plsc.VectorSubcoreMesh(core_axis_name="c", subcore_axis_name="s")

    @functools.partial(
        pl.kernel,
        mesh=mesh,
        out_type=jax.ShapeDtypeStruct((n * _EMBED_DIM,), jnp.float32),
        scratch_types=[
            pltpu.VMEM((16,), jnp.int32),
            pltpu.VMEM((16,), jnp.int32),
            pltpu.VMEM((16,), jnp.int32),
            pltpu.VMEM((16, _ROW_PAD), jnp.float32),
            pltpu.VMEM((16 * _EMBED_DIM,), jnp.float32),
            pltpu.SemaphoreType.DMA,
        ],
    )
    def gather_kernel(src_hbm, tgt_hbm, table_hbm, out_hbm,
                      src_v, tgt_v, idx_v, rows_v, fv_v, sem):
        wid = lax.axis_index("s") * 2 + lax.axis_index("c")

        @pl.when(wid < n_chunks)
        def _():
            base = jnp.minimum(wid * 16, last_base)
            pltpu.sync_copy(src_hbm.at[pl.ds(base, 16)], src_v)
            pltpu.sync_copy(tgt_hbm.at[pl.ds(base, 16)], tgt_v)
            idx_v[...] = lax.bitwise_xor(
                lax.bitwise_xor(src_v[...], tgt_v[...]), 1)
            pltpu.async_copy(table_hbm.at[idx_v], rows_v, sem).wait()
            for i in range(16):
                for c in range(_EMBED_DIM // 16):
                    fv_v[pl.ds(i * _EMBED_DIM + c * 16, 16)] = (
                        rows_v[i, pl.ds(c * 16, 16)])
            pltpu.sync_copy(
                fv_v, out_hbm.at[pl.ds(base * _EMBED_DIM, 16 * _EMBED_DIM)])

    return gather_kernel(src_flat, tgt_flat, table_padded)


def _proj_body(x_ref, w1_ref, w2_ref, fv_ref, b_ref, o_ref, bias_ref):
    j = pl.program_id(1)

    @pl.when(j == 0)
    def _():
        bias_ref[...] = jnp.dot(fv_ref[0], w2_ref[...],
                                preferred_element_type=jnp.float32) + b_ref[...]

    o_ref[...] = (jnp.dot(x_ref[0], w1_ref[...],
                          preferred_element_type=jnp.float32)
                  + bias_ref[...])[None]


def kernel(word_encodings, src_floors, tgt_floors, emb_table, W, b):
    B, S, D = word_encodings.shape
    hist = src_floors.shape[1]
    hv = hist * _EMBED_DIM

    fv_flat = _floor_gather_sc(
        src_floors.astype(jnp.int32).reshape(-1),
        tgt_floors.astype(jnp.int32).reshape(-1),
        jnp.pad(emb_table, ((0, 0), (0, _ROW_PAD - _EMBED_DIM))),
    )
    fv = fv_flat.reshape(B, 1, hv)

    grid = (B, S // _SEQ_TILE)
    out = pl.pallas_call(
        _proj_body,
        grid=grid,
        in_specs=[
            pl.BlockSpec((1, _SEQ_TILE, D), lambda i, j: (i, j, 0)),
            pl.BlockSpec((D, D), lambda i, j: (0, 0)),
            pl.BlockSpec((hv, D), lambda i, j: (0, 0)),
            pl.BlockSpec((1, 1, hv), lambda i, j: (i, 0, 0)),
            pl.BlockSpec((1, D), lambda i, j: (0, 0)),
        ],
        out_specs=pl.BlockSpec((1, _SEQ_TILE, D), lambda i, j: (i, j, 0)),
        out_shape=jax.ShapeDtypeStruct((B, S, D), jnp.float32),
        scratch_shapes=[pltpu.VMEM((1, D), jnp.float32)],
        compiler_params=pltpu.CompilerParams(
            dimension_semantics=("arbitrary", "arbitrary"),
        ),
    )(word_encodings, W[:D], W[D:], fv, b.reshape(1, D))
    return out


# speedup vs baseline: 1.3185x; 1.1229x over previous
import functools

import jax
import jax.numpy as jnp
from jax import lax
from jax.experimental import pallas as pl
from jax.experimental.pallas import tpu as pltpu
from jax.experimental.pallas import tpu_sc as plsc

_EMBED_DIM = 32
_SEQ_TILE = 256


_ROW_PAD = 128


def _floor_gather_sc(src_flat, tgt_flat, table_padded):
    n = src_flat.shape[0]
    n_chunks = (n + 15) // 16
    last_base = n - 16
    mesh = plsc.VectorSubcoreMesh(core_axis_name="c", subcore_axis_name="s")

    @functools.partial(
        pl.kernel,
        mesh=mesh,
        out_type=jax.ShapeDtypeStruct((n * _EMBED_DIM,), jnp.float32),
        scratch_types=[
            pltpu.VMEM((16,), jnp.int32),
            pltpu.VMEM((16,), jnp.int32),
            pltpu.VMEM((16,), jnp.int32),
            pltpu.VMEM((16, _ROW_PAD), jnp.float32),
            pltpu.VMEM((16 * _EMBED_DIM,), jnp.float32),
            pltpu.SemaphoreType.DMA,
        ],
    )
    def gather_kernel(src_hbm, tgt_hbm, table_hbm, out_hbm,
                      src_v, tgt_v, idx_v, rows_v, fv_v, sem):
        wid = lax.axis_index("s") * 2 + lax.axis_index("c")

        @pl.when(wid < n_chunks)
        def _():
            base = jnp.minimum(wid * 16, last_base)
            pltpu.sync_copy(src_hbm.at[pl.ds(base, 16)], src_v)
            pltpu.sync_copy(tgt_hbm.at[pl.ds(base, 16)], tgt_v)
            idx_v[...] = lax.bitwise_xor(
                lax.bitwise_xor(src_v[...], tgt_v[...]), 1)
            pltpu.async_copy(table_hbm.at[idx_v], rows_v, sem).wait()
            for i in range(16):
                for c in range(_EMBED_DIM // 16):
                    fv_v[pl.ds(i * _EMBED_DIM + c * 16, 16)] = (
                        rows_v[i, pl.ds(c * 16, 16)])
            pltpu.sync_copy(
                fv_v, out_hbm.at[pl.ds(base * _EMBED_DIM, 16 * _EMBED_DIM)])

    return gather_kernel(src_flat, tgt_flat, table_padded)


def _proj_body(x_ref, w_ref, fv_ref, b_ref, o_ref, bias_ref, *, d, hv):
    i = pl.program_id(0)
    j = pl.program_id(1)
    nb = bias_ref.shape[0]

    @pl.when((i == 0) & (j == 0))
    def _():
        w2 = w_ref[d:, :]
        for bb in range(nb):
            fvb = fv_ref[pl.ds(bb * hv, hv)].reshape(1, hv)
            bias_ref[pl.ds(bb, 1), :] = (
                jnp.dot(fvb, w2, preferred_element_type=jnp.float32)
                + b_ref[...])

    o_ref[...] = (jnp.dot(x_ref[0], w_ref[:d, :],
                          preferred_element_type=jnp.float32)
                  + bias_ref[pl.ds(i, 1), :])[None]


def kernel(word_encodings, src_floors, tgt_floors, emb_table, W, b):
    B, S, D = word_encodings.shape
    hist = src_floors.shape[1]
    hv = hist * _EMBED_DIM

    fv_flat = _floor_gather_sc(
        src_floors.astype(jnp.int32).reshape(-1),
        tgt_floors.astype(jnp.int32).reshape(-1),
        jnp.pad(emb_table, ((0, 0), (0, _ROW_PAD - _EMBED_DIM))),
    )

    grid = (B, S // _SEQ_TILE)
    out = pl.pallas_call(
        functools.partial(_proj_body, d=D, hv=hv),
        grid=grid,
        in_specs=[
            pl.BlockSpec((1, _SEQ_TILE, D), lambda i, j: (i, j, 0)),
            pl.BlockSpec((D + hv, D), lambda i, j: (0, 0)),
            pl.BlockSpec((B * hv,), lambda i, j: (0,)),
            pl.BlockSpec((1, D), lambda i, j: (0, 0)),
        ],
        out_specs=pl.BlockSpec((1, _SEQ_TILE, D), lambda i, j: (i, j, 0)),
        out_shape=jax.ShapeDtypeStruct((B, S, D), jnp.float32),
        scratch_shapes=[pltpu.VMEM((B, D), jnp.float32)],
        compiler_params=pltpu.CompilerParams(
            dimension_semantics=("arbitrary", "arbitrary"),
        ),
    )(word_encodings, W, fv_flat, b.reshape(1, D))
    return out


# speedup vs baseline: 1.9427x; 1.4734x over previous
import functools

import jax
import jax.numpy as jnp
from jax import lax
from jax.experimental import pallas as pl
from jax.experimental.pallas import tpu as pltpu
from jax.experimental.pallas import tpu_sc as plsc

_EMBED_DIM = 32
_SEQ_TILE = 256


_ROW_PAD = 128


def _floor_gather_sc(src_flat, tgt_flat, table_padded):
    n = src_flat.shape[0]
    n_chunks = (n + 15) // 16
    last_base = n - 16
    mesh = plsc.VectorSubcoreMesh(core_axis_name="c", subcore_axis_name="s")

    @functools.partial(
        pl.kernel,
        mesh=mesh,
        out_type=jax.ShapeDtypeStruct((n * _EMBED_DIM,), jnp.float32),
        scratch_types=[
            pltpu.VMEM((16,), jnp.int32),
            pltpu.VMEM((16,), jnp.int32),
            pltpu.VMEM((16,), jnp.int32),
            pltpu.VMEM((16, _ROW_PAD), jnp.float32),
            pltpu.VMEM((16 * _EMBED_DIM,), jnp.float32),
            pltpu.SemaphoreType.DMA,
        ],
    )
    def gather_kernel(src_hbm, tgt_hbm, table_hbm, out_hbm,
                      src_v, tgt_v, idx_v, rows_v, fv_v, sem):
        wid = lax.axis_index("s") * 2 + lax.axis_index("c")

        @pl.when(wid < n_chunks)
        def _():
            base = jnp.minimum(wid * 16, last_base)
            pltpu.sync_copy(src_hbm.at[pl.ds(base, 16)], src_v)
            pltpu.sync_copy(tgt_hbm.at[pl.ds(base, 16)], tgt_v)
            idx_v[...] = lax.bitwise_xor(
                lax.bitwise_xor(src_v[...], tgt_v[...]), 1)
            pltpu.async_copy(table_hbm.at[idx_v], rows_v, sem).wait()
            for i in range(16):
                for c in range(_EMBED_DIM // 16):
                    fv_v[pl.ds(i * _EMBED_DIM + c * 16, 16)] = (
                        rows_v[i, pl.ds(c * 16, 16)])
            pltpu.sync_copy(
                fv_v, out_hbm.at[pl.ds(base * _EMBED_DIM, 16 * _EMBED_DIM)])

    return gather_kernel(src_flat, tgt_flat, table_padded)


def _proj_body(x_ref, w_ref, fv_ref, b_ref, o_ref, bias_ref, *, d, hv):
    i = pl.program_id(0)
    j = pl.program_id(1)
    nb = bias_ref.shape[0]

    @pl.when((i == 0) & (j == 0))
    def _():
        w2 = w_ref[d:, :]
        for bb in range(nb):
            fvb = fv_ref[pl.ds(bb * hv, hv)].reshape(1, hv)
            bias_ref[pl.ds(bb, 1), :] = (
                jnp.dot(fvb, w2, preferred_element_type=jnp.float32)
                + b_ref[...])

    o_ref[...] = (jnp.dot(x_ref[0], w_ref[:d, :],
                          preferred_element_type=jnp.float32)
                  + bias_ref[pl.ds(i, 1), :])[None]


def kernel(word_encodings, src_floors, tgt_floors, emb_table, W, b):
    B, S, D = word_encodings.shape
    hist = src_floors.shape[1]
    hv = hist * _EMBED_DIM

    same = (src_floors == tgt_floors)
    fv_flat = jnp.where(same[..., None], emb_table[1], emb_table[0]).reshape(-1)

    grid = (B, S // _SEQ_TILE)
    out = pl.pallas_call(
        functools.partial(_proj_body, d=D, hv=hv),
        grid=grid,
        in_specs=[
            pl.BlockSpec((1, _SEQ_TILE, D), lambda i, j: (i, j, 0)),
            pl.BlockSpec((D + hv, D), lambda i, j: (0, 0)),
            pl.BlockSpec((B * hv,), lambda i, j: (0,)),
            pl.BlockSpec((1, D), lambda i, j: (0, 0)),
        ],
        out_specs=pl.BlockSpec((1, _SEQ_TILE, D), lambda i, j: (i, j, 0)),
        out_shape=jax.ShapeDtypeStruct((B, S, D), jnp.float32),
        scratch_shapes=[pltpu.VMEM((B, D), jnp.float32)],
        compiler_params=pltpu.CompilerParams(
            dimension_semantics=("arbitrary", "arbitrary"),
        ),
    )(word_encodings, W, fv_flat, b.reshape(1, D))
    return out
